# Initial kernel scaffold; baseline (speedup 1.0000x reference)
import jax
import jax.numpy as jnp
from jax.experimental import pallas as pl


def kernel(x, adj, weight):
    raise NotImplementedError("write your pallas kernel here")


# speedup vs baseline: 1.0063x; 1.0063x over previous
import jax
import jax.numpy as jnp
from jax.experimental import pallas as pl
from jax.experimental.pallas import tpu as pltpu


def _round_up(v, m):
    return ((v + m - 1) // m) * m


def _gcn_slab_kernel(adj_ref, x_ref, w_ref, out_ref):
    ax = jnp.dot(adj_ref[...], x_ref[...], preferred_element_type=jnp.float32)
    out_ref[...] = jnp.dot(
        ax, w_ref[...], preferred_element_type=jnp.float32
    ).astype(out_ref.dtype)


@jax.jit
def kernel(x, adj, weight):
    n, f_in = x.shape
    f_out = weight.shape[1]

    fi = _round_up(f_in, 128)
    fo = _round_up(f_out, 128)

    n_lane = _round_up(n, 128)
    tm = 512 if n_lane >= 512 else n_lane
    while tm > 128 and 2 * tm * n_lane * 4 > (36 << 20):
        tm //= 2
    n_pad = _round_up(n, tm)

    if (n_pad, fi) != (n, f_in):
        x = jnp.pad(x, ((0, n_pad - n), (0, fi - f_in)))
    if n_pad != n:
        adj = jnp.pad(adj, ((0, n_pad - n), (0, n_pad - n)))
    if (fi, fo) != weight.shape:
        weight = jnp.pad(weight, ((0, fi - f_in), (0, fo - f_out)))

    out = pl.pallas_call(
        _gcn_slab_kernel,
        out_shape=jax.ShapeDtypeStruct((n_pad, fo), x.dtype),
        grid=(n_pad // tm,),
        in_specs=[
            pl.BlockSpec((tm, n_pad), lambda i: (i, 0)),
            pl.BlockSpec((n_pad, fi), lambda i: (0, 0)),
            pl.BlockSpec((fi, fo), lambda i: (0, 0)),
        ],
        out_specs=pl.BlockSpec((tm, fo), lambda i: (i, 0)),
        compiler_params=pltpu.CompilerParams(
            dimension_semantics=("parallel",),
            vmem_limit_bytes=52 * 1024 * 1024,
        ),
    )(adj, x, weight)

    if (n_pad, fo) != (n, f_out):
        out = out[:n, :f_out]
    return out
